# baseline (speedup 1.0000x reference)
import functools

import jax
import jax.numpy as jnp
from jax import lax
from jax.experimental import pallas as pl
from jax.experimental.pallas import tpu as pltpu
from jax.experimental.pallas import tpu_sc as plsc

VOCAB_SIZE = 1000000
H = 32
B = 16384

_info = plsc.get_sparse_core_info()
_NC, _NS, _L = _info.num_cores, _info.num_subcores, _info.num_lanes
_NW = _NC * _NS
_BPW = B // _NW
_K = 16

_mesh = plsc.VectorSubcoreMesh(core_axis_name="c", subcore_axis_name="s")


@functools.partial(
    pl.kernel,
    mesh=_mesh,
    out_type=jax.ShapeDtypeStruct((B, H), jnp.float32),
    scratch_types=[
        pltpu.SMEM((_BPW,), jnp.int32),
        pltpu.VMEM_SHARED((B,), jnp.int32),
        pltpu.VMEM((_BPW, H), jnp.float32),
        pltpu.SemaphoreType.DMA,
    ],
)
def _sc_gather(table_hbm, idx_hbm, out_hbm, idx_s, idx_sp, rows_v, sem):
    wid = lax.axis_index("s") * _NC + lax.axis_index("c")
    base = wid * _BPW
    pltpu.sync_copy(idx_hbm.at[pl.ds(base, _BPW)], idx_sp.at[pl.ds(base, _BPW)])
    pltpu.sync_copy(idx_sp.at[pl.ds(base, _BPW)], idx_s)

    def group(g, _):
        descs = []
        for j in range(_K):
            i = g * _K + j
            row = idx_s[i]
            descs.append(
                pltpu.async_copy(
                    table_hbm.at[pl.ds(row, 1)], rows_v.at[pl.ds(i, 1)], sem
                )
            )
        for d in descs:
            d.wait()
        return ()

    lax.fori_loop(0, _BPW // _K, group, (), unroll=False)
    pltpu.sync_copy(rows_v, out_hbm.at[pl.ds(base, _BPW)])


def _mm_body(e_ref, w_ref, b_ref, o_ref):
    o_ref[...] = (
        jnp.dot(e_ref[...], w_ref[...], preferred_element_type=jnp.float32)
        + b_ref[...]
    )


def kernel(x, table, W, b):
    idx = x.reshape(B).astype(jnp.int32)
    e = _sc_gather(table, idx)
    blk = 2048
    out = pl.pallas_call(
        _mm_body,
        out_shape=jax.ShapeDtypeStruct((B, H), jnp.float32),
        grid=(B // blk,),
        in_specs=[
            pl.BlockSpec((blk, H), lambda i: (i, 0)),
            pl.BlockSpec((H, H), lambda i: (0, 0)),
            pl.BlockSpec((1, H), lambda i: (0, 0)),
        ],
        out_specs=pl.BlockSpec((blk, H), lambda i: (i, 0)),
    )(e, W, b.reshape(1, H))
    return out


# speedup vs baseline: 1.0428x; 1.0428x over previous
import functools

import jax
import jax.numpy as jnp
from jax import lax
from jax.experimental import pallas as pl
from jax.experimental.pallas import tpu as pltpu
from jax.experimental.pallas import tpu_sc as plsc

VOCAB_SIZE = 1000000
H = 32
B = 16384

_info = plsc.get_sparse_core_info()
_NC, _NS, _L = _info.num_cores, _info.num_subcores, _info.num_lanes
_NW = _NC * _NS
_BPW = B // _NW
_K = 16

_mesh = plsc.VectorSubcoreMesh(core_axis_name="c", subcore_axis_name="s")


@functools.partial(
    pl.kernel,
    mesh=_mesh,
    out_type=jax.ShapeDtypeStruct((B, H), jnp.float32),
    scratch_types=[
        pltpu.SMEM((_BPW,), jnp.int32),
        pltpu.VMEM_SHARED((B,), jnp.int32),
        pltpu.VMEM((_BPW, H), jnp.float32),
        pltpu.SemaphoreType.DMA,
    ],
)
def _sc_gather(table_hbm, idx_hbm, out_hbm, idx_s, idx_sp, rows_v, sem):
    wid = lax.axis_index("s") * _NC + lax.axis_index("c")
    base = wid * _BPW
    pltpu.sync_copy(idx_hbm.at[pl.ds(base, _BPW)], idx_sp.at[pl.ds(base, _BPW)])
    pltpu.sync_copy(idx_sp.at[pl.ds(base, _BPW)], idx_s)

    n_groups = _BPW // _K

    def issue(g):
        for j in range(_K):
            i = g * _K + j
            pltpu.async_copy(
                table_hbm.at[pl.ds(idx_s[i], 1)], rows_v.at[pl.ds(i, 1)], sem
            )

    def drain(g):
        pltpu.make_async_copy(
            table_hbm.at[pl.ds(0, _K)], rows_v.at[pl.ds(g * _K, _K)], sem
        ).wait()

    issue(0)
    issue(1)

    def body(g, _):
        issue(g)
        drain(g - 2)
        return ()

    lax.fori_loop(2, n_groups, body, (), unroll=False)
    drain(n_groups - 2)
    drain(n_groups - 1)
    pltpu.sync_copy(rows_v, out_hbm.at[pl.ds(base, _BPW)])


def _mm_body(e_ref, w_ref, b_ref, o_ref):
    o_ref[...] = (
        jnp.dot(e_ref[...], w_ref[...], preferred_element_type=jnp.float32)
        + b_ref[...]
    )


def kernel(x, table, W, b):
    idx = x.reshape(B).astype(jnp.int32)
    e = _sc_gather(table, idx)
    blk = 2048
    out = pl.pallas_call(
        _mm_body,
        out_shape=jax.ShapeDtypeStruct((B, H), jnp.float32),
        grid=(B // blk,),
        in_specs=[
            pl.BlockSpec((blk, H), lambda i: (i, 0)),
            pl.BlockSpec((H, H), lambda i: (0, 0)),
            pl.BlockSpec((1, H), lambda i: (0, 0)),
        ],
        out_specs=pl.BlockSpec((blk, H), lambda i: (i, 0)),
    )(e, W, b.reshape(1, H))
    return out


# speedup vs baseline: 1.0568x; 1.0135x over previous
import functools

import jax
import jax.numpy as jnp
from jax import lax
from jax.experimental import pallas as pl
from jax.experimental.pallas import tpu as pltpu
from jax.experimental.pallas import tpu_sc as plsc

VOCAB_SIZE = 1000000
H = 32
B = 16384

_info = plsc.get_sparse_core_info()
_NC, _NS, _L = _info.num_cores, _info.num_subcores, _info.num_lanes
_NW = _NC * _NS
_BPW = B // _NW
_K = 32
_LAG = 4

_mesh = plsc.VectorSubcoreMesh(core_axis_name="c", subcore_axis_name="s")


@functools.partial(
    pl.kernel,
    mesh=_mesh,
    out_type=jax.ShapeDtypeStruct((B, H), jnp.float32),
    scratch_types=[
        pltpu.SMEM((_BPW,), jnp.int32),
        pltpu.VMEM_SHARED((B,), jnp.int32),
        pltpu.VMEM((_BPW, H), jnp.float32),
        pltpu.SemaphoreType.DMA,
    ],
)
def _sc_gather(table_hbm, idx_hbm, out_hbm, idx_s, idx_sp, rows_v, sem):
    wid = lax.axis_index("s") * _NC + lax.axis_index("c")
    base = wid * _BPW
    pltpu.sync_copy(idx_hbm.at[pl.ds(base, _BPW)], idx_sp.at[pl.ds(base, _BPW)])
    pltpu.sync_copy(idx_sp.at[pl.ds(base, _BPW)], idx_s)

    n_groups = _BPW // _K

    def issue(g):
        for j in range(_K):
            i = g * _K + j
            pltpu.async_copy(
                table_hbm.at[pl.ds(idx_s[i], 1)], rows_v.at[pl.ds(i, 1)], sem
            )

    def drain(g):
        pltpu.make_async_copy(
            table_hbm.at[pl.ds(0, _K)], rows_v.at[pl.ds(g * _K, _K)], sem
        ).wait()

    for g in range(_LAG):
        issue(g)

    def body(g, _):
        issue(g)
        drain(g - _LAG)
        return ()

    lax.fori_loop(_LAG, n_groups, body, (), unroll=False)
    for g in range(n_groups - _LAG, n_groups):
        drain(g)
    pltpu.sync_copy(rows_v, out_hbm.at[pl.ds(base, _BPW)])


def _mm_body(e_ref, w_ref, b_ref, o_ref):
    o_ref[...] = (
        jnp.dot(e_ref[...], w_ref[...], preferred_element_type=jnp.float32)
        + b_ref[...]
    )


def kernel(x, table, W, b):
    idx = x.reshape(B).astype(jnp.int32)
    e = _sc_gather(table, idx)
    blk = 2048
    out = pl.pallas_call(
        _mm_body,
        out_shape=jax.ShapeDtypeStruct((B, H), jnp.float32),
        grid=(B // blk,),
        in_specs=[
            pl.BlockSpec((blk, H), lambda i: (i, 0)),
            pl.BlockSpec((H, H), lambda i: (0, 0)),
            pl.BlockSpec((1, H), lambda i: (0, 0)),
        ],
        out_specs=pl.BlockSpec((blk, H), lambda i: (i, 0)),
    )(e, W, b.reshape(1, H))
    return out
